# Initial kernel scaffold; baseline (speedup 1.0000x reference)
import jax
import jax.numpy as jnp
from jax.experimental import pallas as pl


def kernel(q, k, v):
    raise NotImplementedError("write your pallas kernel here")


# speedup vs baseline: 20.6771x; 20.6771x over previous
import jax
import jax.numpy as jnp
from jax.experimental import pallas as pl

_VMASK = (1 << 17) - 1
_TQ = 64


def _body(q_ref, k_ref, p_ref, o_ref):
    j = pl.program_id(1)
    qt = q_ref[0]
    krow = k_ref[0]
    prow = p_ref[0]
    S = krow.shape[1]
    t_global = j * _TQ + jax.lax.broadcasted_iota(jnp.int32, (_TQ, S), 0)
    tp = jax.lax.broadcasted_iota(jnp.int32, (_TQ, S), 1)
    hit = (qt == krow) & (tp < t_global)
    m = jnp.max(jnp.where(hit, prow, 0), axis=1, keepdims=True)
    o_ref[0] = m & _VMASK


def kernel(q, k, v):
    B, S = q.shape
    q32 = q.astype(jnp.int32).reshape(B, S, 1)
    k32 = k.astype(jnp.int32).reshape(B, 1, S)
    packed = (((jnp.arange(S, dtype=jnp.int32) + 1) << 17)
              | v.astype(jnp.int32)).reshape(B, 1, S)

    out = pl.pallas_call(
        _body,
        grid=(B, S // _TQ),
        in_specs=[
            pl.BlockSpec((1, _TQ, 1), lambda b, j: (b, j, j * 0)),
            pl.BlockSpec((1, 1, S), lambda b, j: (b, j * 0, j * 0)),
            pl.BlockSpec((1, 1, S), lambda b, j: (b, j * 0, j * 0)),
        ],
        out_specs=pl.BlockSpec((1, _TQ, 1), lambda b, j: (b, j, j * 0)),
        out_shape=jax.ShapeDtypeStruct((B, S, 1), jnp.int32),
    )(q32, k32, packed)
    return out.reshape(B, S).astype(q.dtype)


# speedup vs baseline: 216.6395x; 10.4773x over previous
import functools

import jax
import jax.numpy as jnp
from jax import lax
from jax.experimental import pallas as pl
from jax.experimental.pallas import tpu as pltpu
from jax.experimental.pallas import tpu_sc as plsc

_NC = 2
_NS = 16
_L = 16
_VOCAB = 100000


def _sc_body(q_hbm, k_hbm, v_hbm, out_hbm, tab, qv, kv, vv, ov):
    B, S = q_hbm.shape
    nchunks = S // _L
    rows_per_w = B // (_NC * _NS)
    wid = lax.axis_index("s") * _NC + lax.axis_index("c")
    lane = lax.iota(jnp.int32, _L)
    zero16 = jnp.zeros((_L,), jnp.int32)

    lL = jnp.int32(_L)
    for r in range(rows_per_w):
        row = wid * jnp.int32(rows_per_w) + jnp.int32(r)
        pltpu.sync_copy(q_hbm.at[row], qv)
        pltpu.sync_copy(k_hbm.at[row], kv)
        pltpu.sync_copy(v_hbm.at[row], vv)

        def zero_body(c, carry):
            base = c * lL
            plsc.store_scatter(tab, [qv[pl.ds(base, _L)]], zero16)
            plsc.store_scatter(tab, [kv[pl.ds(base, _L)]], zero16)
            return carry

        lax.fori_loop(jnp.int32(0), jnp.int32(nchunks), zero_body,
                      jnp.int32(0), unroll=False)

        def chunk_body(c, carry):
            base = c * lL
            qc = qv[pl.ds(base, _L)]
            kc = kv[pl.ds(base, _L)]
            vc = vv[pl.ds(base, _L)]
            outc = plsc.load_gather(tab, [qc])
            dup = qc != qc
            for j in range(_L):
                idxj = jnp.full((_L,), base + jnp.int32(j), jnp.int32)
                kj = plsc.load_gather(kv, [idxj])
                vj = plsc.load_gather(vv, [idxj])
                outc = jnp.where((qc == kj) & (lane > j), vj, outc)
                dup = dup | ((kc == kj) & (lane < j))
            plsc.store_scatter(tab, [kc], vc, mask=jnp.logical_not(dup))
            ov[pl.ds(base, _L)] = outc
            return carry

        lax.fori_loop(jnp.int32(0), jnp.int32(nchunks), chunk_body,
                      jnp.int32(0), unroll=False)
        pltpu.sync_copy(ov, out_hbm.at[row])


def kernel(q, k, v):
    B, S = q.shape
    q32 = q.astype(jnp.int32)
    k32 = k.astype(jnp.int32)
    v32 = v.astype(jnp.int32)

    mesh = plsc.VectorSubcoreMesh(core_axis_name="c", subcore_axis_name="s")
    run = functools.partial(
        pl.kernel,
        out_type=jax.ShapeDtypeStruct((B, S), jnp.int32),
        mesh=mesh,
        scratch_types=[
            pltpu.VMEM((_VOCAB,), jnp.int32),
            pltpu.VMEM((S,), jnp.int32),
            pltpu.VMEM((S,), jnp.int32),
            pltpu.VMEM((S,), jnp.int32),
            pltpu.VMEM((S,), jnp.int32),
        ],
        compiler_params=pltpu.CompilerParams(needs_layout_passes=False),
    )(_sc_body)
    out = run(q32, k32, v32)
    return out.astype(q.dtype)
